# baseline (speedup 1.0000x reference)
import functools

import jax
import jax.numpy as jnp
from jax import lax
from jax.experimental import pallas as pl
from jax.experimental.pallas import tpu as pltpu
from jax.experimental.pallas import tpu_sc as plsc

N = 100000
E = 1600000
D = 32
H = 16

NSUB = 16
KB = 3
CHUNK = KB * 128

NP = 100352
TR = NP // NSUB
ZR = 128
EP = 1609728
G = EP // 128
GT = G // NSUB
NCHUNK = GT // KB

BR = 2048
_MESH = plsc.VectorSubcoreMesh(
    core_axis_name="c", subcore_axis_name="s", num_cores=2, num_subcores=NSUB
)


def _agg_body(gidx_hbm, sidx_hbm, table_hbm, out_hbm,
              acc_sh, gi_v, si_v, rows_v, zb_v, sem0, sem1):
    c = lax.axis_index("c")
    s = lax.axis_index("s")
    sems = (sem0, sem1)

    def _zero(j, carry):
        zb_v[j, :] = jnp.zeros((16,), jnp.float32)
        return carry

    lax.fori_loop(0, ZR, _zero, 0)
    r0 = s * TR
    for t in range(TR // ZR):
        pltpu.sync_copy(zb_v, acc_sh.at[pl.ds(r0 + t * ZR, ZR)])
    plsc.subcore_barrier()

    cvec = jnp.full((16,), c, jnp.int32)

    def _fire(b, k):
        g0 = s * GT + k * KB
        pltpu.sync_copy(gidx_hbm.at[pl.ds(g0, KB)], gi_v.at[b])
        pltpu.sync_copy(sidx_hbm.at[pl.ds(g0, KB)], si_v.at[b])
        for j in range(KB):
            for q in range(8):
                v = gi_v[b, j, pl.ds(q * 16, 16)]
                gi_v[b, j, pl.ds(q * 16, 16)] = v + v + cvec
        for j in range(KB):
            pltpu.async_copy(table_hbm.at[gi_v.at[b, j]],
                             rows_v.at[b, pl.ds(j * 128, 128)], sems[b])

    def _drain(b):
        for j in range(KB):
            pltpu.make_async_copy(table_hbm.at[gi_v.at[b, j]],
                                  rows_v.at[b, pl.ds(j * 128, 128)],
                                  sems[b]).wait()

    def _scatter(b):
        for j in range(KB):
            pltpu.sync_copy(rows_v.at[b, pl.ds(j * 128, 128)],
                            acc_sh.at[si_v.at[b, j]], add=True)

    _fire(0, 0)

    def _pair(k2, carry):
        k = k2 * 2
        _fire(1, k + 1)
        _drain(0)
        _scatter(0)

        @pl.when(k + 2 < NCHUNK)
        def _():
            _fire(0, k + 2)

        _drain(1)
        _scatter(1)
        return carry

    lax.fori_loop(0, NCHUNK // 2, _pair, 0)
    plsc.subcore_barrier()

    @pl.when(c == 0)
    def _():
        pltpu.sync_copy(acc_sh.at[pl.ds(r0, TR)],
                        out_hbm.at[pl.ds(r0, TR), pl.ds(0, 16)])

    @pl.when(c == 1)
    def _():
        pltpu.sync_copy(acc_sh.at[pl.ds(r0, TR)],
                        out_hbm.at[pl.ds(r0, TR), pl.ds(16, 16)])


_agg_call = functools.partial(
    pl.kernel,
    out_type=jax.ShapeDtypeStruct((NP, D), jnp.float32),
    mesh=_MESH,
    scratch_types=[
        pltpu.VMEM_SHARED((NP, H), jnp.float32),
        pltpu.VMEM((2, KB, 128), jnp.int32),
        pltpu.VMEM((2, KB, 128), jnp.int32),
        pltpu.VMEM((2, CHUNK, H), jnp.float32),
        pltpu.VMEM((ZR, H), jnp.float32),
        pltpu.SemaphoreType.DMA,
        pltpu.SemaphoreType.DMA,
    ],
    compiler_params=pltpu.CompilerParams(use_tc_tiling_on_sc=False),
)(_agg_body)


def _cnt_body(su_hbm, sd_hbm, out_hbm, acc_sh, si_v, ones_v, zb_v,
              sem0, sem1):
    c = lax.axis_index("c")
    s = lax.axis_index("s")

    def _zero(j, carry):
        zb_v[j, :] = jnp.zeros((16,), jnp.float32)
        return carry

    lax.fori_loop(0, ZR, _zero, 0)

    def _one(j, carry):
        ones_v[j, :] = jnp.ones((16,), jnp.float32)
        return carry

    lax.fori_loop(0, 128, _one, 0)
    r0 = s * TR
    for t in range(TR // ZR):
        pltpu.sync_copy(zb_v, acc_sh.at[pl.ds(r0 + t * ZR, ZR)])
    plsc.subcore_barrier()

    sems = (sem0, sem1)

    def _fire(b, k):
        g0 = s * GT + k * KB

        @pl.when(c == 0)
        def _():
            pltpu.sync_copy(su_hbm.at[pl.ds(g0, KB)], si_v.at[b])

        @pl.when(c == 1)
        def _():
            pltpu.sync_copy(sd_hbm.at[pl.ds(g0, KB)], si_v.at[b])

        for j in range(KB):
            pltpu.async_copy(ones_v, acc_sh.at[si_v.at[b, j]], sems[b],
                             add=True)

    def _drain(b):
        for j in range(KB):
            pltpu.make_async_copy(ones_v, acc_sh.at[si_v.at[b, j]],
                                  sems[b]).wait()

    _fire(0, 0)

    def _pair(k2, carry):
        k = k2 * 2
        _fire(1, k + 1)
        _drain(0)

        @pl.when(k + 2 < NCHUNK)
        def _():
            _fire(0, k + 2)

        _drain(1)
        return carry

    lax.fori_loop(0, NCHUNK // 2, _pair, 0)
    plsc.subcore_barrier()

    @pl.when(c == 0)
    def _():
        pltpu.sync_copy(acc_sh.at[pl.ds(r0, TR)],
                        out_hbm.at[pl.ds(r0, TR), pl.ds(0, 16)])

    @pl.when(c == 1)
    def _():
        pltpu.sync_copy(acc_sh.at[pl.ds(r0, TR)],
                        out_hbm.at[pl.ds(r0, TR), pl.ds(16, 16)])


_cnt_call = functools.partial(
    pl.kernel,
    out_type=jax.ShapeDtypeStruct((NP, D), jnp.float32),
    mesh=_MESH,
    scratch_types=[
        pltpu.VMEM_SHARED((NP, H), jnp.float32),
        pltpu.VMEM((2, KB, 128), jnp.int32),
        pltpu.VMEM((128, H), jnp.float32),
        pltpu.VMEM((ZR, H), jnp.float32),
        pltpu.SemaphoreType.DMA,
        pltpu.SemaphoreType.DMA,
    ],
    compiler_params=pltpu.CompilerParams(use_tc_tiling_on_sc=False),
)(_cnt_body)


def _dense_body(agg_ref, cnt_ref, xd_ref, wl_ref, bl_ref, wr_ref, out_ref,
                *, col):
    cnt = jnp.maximum(cnt_ref[:, col:col + 1], 1.0)
    agg = agg_ref[...] / cnt
    o = (jnp.dot(agg, wl_ref[...], preferred_element_type=jnp.float32)
         + bl_ref[...]
         + jnp.dot(xd_ref[...], wr_ref[...], preferred_element_type=jnp.float32))
    nrm = jnp.sqrt(jnp.sum(o * o, axis=-1, keepdims=True))
    h = o / jnp.maximum(nrm, 1e-12)
    out_ref[...] = jnp.maximum(h, 0.0)


def _dense_final_body(agg_ref, cnt_ref, xd_ref, wl_ref, bl_ref, wr_ref,
                      wo_ref, bo_ref, out_ref, *, col):
    cnt = jnp.maximum(cnt_ref[:, col:col + 1], 1.0)
    agg = agg_ref[...] / cnt
    o = (jnp.dot(agg, wl_ref[...], preferred_element_type=jnp.float32)
         + bl_ref[...]
         + jnp.dot(xd_ref[...], wr_ref[...], preferred_element_type=jnp.float32))
    nrm = jnp.sqrt(jnp.sum(o * o, axis=-1, keepdims=True))
    h = jnp.maximum(o / jnp.maximum(nrm, 1e-12), 0.0)
    e = jnp.dot(h, wo_ref[...], preferred_element_type=jnp.float32) + bo_ref[...]
    nrm2 = jnp.sqrt(jnp.sum(e * e, axis=-1, keepdims=True))
    out_ref[...] = e / jnp.maximum(nrm2, 1e-12)


_ROW_SPEC = pl.BlockSpec((BR, D), lambda i: (i, 0))
_W_SPEC = pl.BlockSpec((D, D), lambda i: (0, 0))
_B_SPEC = pl.BlockSpec((1, D), lambda i: (0, 0))


def _dense(agg, cnt, xd, wl, bl, wr, col):
    return pl.pallas_call(
        functools.partial(_dense_body, col=col),
        grid=(NP // BR,),
        in_specs=[_ROW_SPEC, _ROW_SPEC, _ROW_SPEC, _W_SPEC, _B_SPEC, _W_SPEC],
        out_specs=_ROW_SPEC,
        out_shape=jax.ShapeDtypeStruct((NP, D), jnp.float32),
    )(agg, cnt, xd, wl, bl.reshape(1, D), wr)


def _dense_final(agg, cnt, xd, wl, bl, wr, wo, bo, col):
    return pl.pallas_call(
        functools.partial(_dense_final_body, col=col),
        grid=(NP // BR,),
        in_specs=[_ROW_SPEC, _ROW_SPEC, _ROW_SPEC, _W_SPEC, _B_SPEC, _W_SPEC,
                  _W_SPEC, _B_SPEC],
        out_specs=_ROW_SPEC,
        out_shape=jax.ShapeDtypeStruct((N, D), jnp.float32),
    )(agg, cnt, xd, wl, bl.reshape(1, D), wr, wo, bo.reshape(1, D))


def kernel(x_user, x_item, edge_index,
           Wl_ui0, bl_ui0, Wr_ui0, Wl_iu0, bl_iu0, Wr_iu0,
           Wl_ui1, bl_ui1, Wr_ui1, Wl_iu1, bl_iu1, Wr_iu1,
           Wu, bu, Wi, bi):
    src = edge_index[0].astype(jnp.int32)
    dst = edge_index[1].astype(jnp.int32)
    pad = jnp.full((EP - E,), N, jnp.int32)
    src_p = jnp.concatenate([src, pad]).reshape(G, 128)
    dst_p = jnp.concatenate([dst, pad]).reshape(G, 128)
    xu = jnp.pad(x_user, ((0, NP - N), (0, 0)))
    xi = jnp.pad(x_item, ((0, NP - N), (0, 0)))

    cnt = _cnt_call(src_p, dst_p)

    agg_i0 = _agg_call(src_p, dst_p, xu.reshape(2 * NP, H))
    agg_u0 = _agg_call(dst_p, src_p, xi.reshape(2 * NP, H))
    item_h = _dense(agg_i0, cnt, xi, Wl_ui0, bl_ui0, Wr_ui0, 16)
    user_h = _dense(agg_u0, cnt, xu, Wl_iu0, bl_iu0, Wr_iu0, 0)

    agg_i1 = _agg_call(src_p, dst_p, user_h.reshape(2 * NP, H))
    agg_u1 = _agg_call(dst_p, src_p, item_h.reshape(2 * NP, H))
    item_emb = _dense_final(agg_i1, cnt, item_h, Wl_ui1, bl_ui1, Wr_ui1,
                            Wi, bi, 16)
    user_emb = _dense_final(agg_u1, cnt, user_h, Wl_iu1, bl_iu1, Wr_iu1,
                            Wu, bu, 0)
    return (user_emb, item_emb)


# speedup vs baseline: 1.4656x; 1.4656x over previous
import functools

import jax
import jax.numpy as jnp
from jax import lax
from jax.experimental import pallas as pl
from jax.experimental.pallas import tpu as pltpu
from jax.experimental.pallas import tpu_sc as plsc

N = 100000
E = 1600000
D = 32
H = 16

NSUB = 16
KB = 4
CHUNK = KB * 128

NP = 100352
TR = NP // NSUB
ZR = 128
EP = 1605632
G = EP // 128
GT = G // NSUB
NCHUNK = GT // KB

BR = 2048
_MESH = plsc.VectorSubcoreMesh(
    core_axis_name="c", subcore_axis_name="s", num_cores=2, num_subcores=NSUB
)


def _agg_body(gidx_hbm, sidx_hbm, table_hbm, out_hbm,
              acc_sh, gi_v, si_v, rows_v, zb_v, sem0, sem1, semi0, semi1):
    c = lax.axis_index("c")
    s = lax.axis_index("s")
    sems = (sem0, sem1)
    semi = (semi0, semi1)

    def _zero(j, carry):
        zb_v[j, :] = jnp.zeros((16,), jnp.float32)
        return carry

    lax.fori_loop(0, ZR, _zero, 0)
    r0 = s * TR
    for t in range(TR // ZR):
        pltpu.sync_copy(zb_v, acc_sh.at[pl.ds(r0 + t * ZR, ZR)])
    plsc.subcore_barrier()

    cvec = jnp.full((16,), c, jnp.int32)

    def _load_idx(b, k):
        g0 = s * GT + k * KB
        pltpu.async_copy(gidx_hbm.at[pl.ds(g0, KB)], gi_v.at[b], semi[b])
        pltpu.async_copy(sidx_hbm.at[pl.ds(g0, KB)], si_v.at[b], semi[b])

    def _drain_idx(b):
        pltpu.make_async_copy(gidx_hbm.at[pl.ds(0, KB)], gi_v.at[b],
                              semi[b]).wait()
        pltpu.make_async_copy(sidx_hbm.at[pl.ds(0, KB)], si_v.at[b],
                              semi[b]).wait()

    def _transform_fire(b):
        for j in range(KB):
            for q in range(8):
                v = gi_v[b, j, pl.ds(q * 16, 16)]
                gi_v[b, j, pl.ds(q * 16, 16)] = v + v + cvec
        for j in range(KB):
            pltpu.async_copy(table_hbm.at[gi_v.at[b, j]],
                             rows_v.at[b, pl.ds(j * 128, 128)], sems[b])

    def _fire(b, k):
        _load_idx(b, k)
        _drain_idx(b)
        _transform_fire(b)

    def _drain(b):
        for j in range(KB):
            pltpu.make_async_copy(table_hbm.at[gi_v.at[b, j]],
                                  rows_v.at[b, pl.ds(j * 128, 128)],
                                  sems[b]).wait()

    def _scatter(b):
        for j in range(KB):
            pltpu.sync_copy(rows_v.at[b, pl.ds(j * 128, 128)],
                            acc_sh.at[si_v.at[b, j]], add=True)

    _fire(0, 0)
    _fire(1, 1)

    def _pair(k2, carry):
        k = k2 * 2
        _drain(0)

        @pl.when(k + 2 < NCHUNK)
        def _():
            _load_idx(0, k + 2)

        _scatter(0)

        @pl.when(k + 2 < NCHUNK)
        def _():
            _drain_idx(0)
            _transform_fire(0)

        _drain(1)

        @pl.when(k + 3 < NCHUNK)
        def _():
            _load_idx(1, k + 3)

        _scatter(1)

        @pl.when(k + 3 < NCHUNK)
        def _():
            _drain_idx(1)
            _transform_fire(1)

        return carry

    lax.fori_loop(0, NCHUNK // 2, _pair, 0)
    plsc.subcore_barrier()

    @pl.when(c == 0)
    def _():
        pltpu.sync_copy(acc_sh.at[pl.ds(r0, TR)],
                        out_hbm.at[pl.ds(r0, TR), pl.ds(0, 16)])

    @pl.when(c == 1)
    def _():
        pltpu.sync_copy(acc_sh.at[pl.ds(r0, TR)],
                        out_hbm.at[pl.ds(r0, TR), pl.ds(16, 16)])


_agg_call = functools.partial(
    pl.kernel,
    out_type=jax.ShapeDtypeStruct((NP, D), jnp.float32),
    mesh=_MESH,
    scratch_types=[
        pltpu.VMEM_SHARED((NP, H), jnp.float32),
        pltpu.VMEM((2, KB, 128), jnp.int32),
        pltpu.VMEM((2, KB, 128), jnp.int32),
        pltpu.VMEM((2, CHUNK, H), jnp.float32),
        pltpu.VMEM((ZR, H), jnp.float32),
        pltpu.SemaphoreType.DMA,
        pltpu.SemaphoreType.DMA,
        pltpu.SemaphoreType.DMA,
        pltpu.SemaphoreType.DMA,
    ],
    compiler_params=pltpu.CompilerParams(use_tc_tiling_on_sc=False),
)(_agg_body)


def _cnt_body(su_hbm, sd_hbm, out_hbm, acc_sh, si_v, ones_v, zb_v,
              sem0, sem1):
    c = lax.axis_index("c")
    s = lax.axis_index("s")

    def _zero(j, carry):
        zb_v[j, :] = jnp.zeros((16,), jnp.float32)
        return carry

    lax.fori_loop(0, ZR, _zero, 0)

    def _one(j, carry):
        ones_v[j, :] = jnp.ones((16,), jnp.float32)
        return carry

    lax.fori_loop(0, 128, _one, 0)
    r0 = s * TR
    for t in range(TR // ZR):
        pltpu.sync_copy(zb_v, acc_sh.at[pl.ds(r0 + t * ZR, ZR)])
    plsc.subcore_barrier()

    sems = (sem0, sem1)

    def _fire(b, k):
        g0 = s * GT + k * KB

        @pl.when(c == 0)
        def _():
            pltpu.sync_copy(su_hbm.at[pl.ds(g0, KB)], si_v.at[b])

        @pl.when(c == 1)
        def _():
            pltpu.sync_copy(sd_hbm.at[pl.ds(g0, KB)], si_v.at[b])

        for j in range(KB):
            pltpu.async_copy(ones_v, acc_sh.at[si_v.at[b, j]], sems[b],
                             add=True)

    def _drain(b):
        for j in range(KB):
            pltpu.make_async_copy(ones_v, acc_sh.at[si_v.at[b, j]],
                                  sems[b]).wait()

    _fire(0, 0)

    def _pair(k2, carry):
        k = k2 * 2
        _fire(1, k + 1)
        _drain(0)

        @pl.when(k + 2 < NCHUNK)
        def _():
            _fire(0, k + 2)

        _drain(1)
        return carry

    lax.fori_loop(0, NCHUNK // 2, _pair, 0)
    plsc.subcore_barrier()

    @pl.when(c == 0)
    def _():
        pltpu.sync_copy(acc_sh.at[pl.ds(r0, TR)],
                        out_hbm.at[pl.ds(r0, TR), pl.ds(0, 16)])

    @pl.when(c == 1)
    def _():
        pltpu.sync_copy(acc_sh.at[pl.ds(r0, TR)],
                        out_hbm.at[pl.ds(r0, TR), pl.ds(16, 16)])


_cnt_call = functools.partial(
    pl.kernel,
    out_type=jax.ShapeDtypeStruct((NP, D), jnp.float32),
    mesh=_MESH,
    scratch_types=[
        pltpu.VMEM_SHARED((NP, H), jnp.float32),
        pltpu.VMEM((2, KB, 128), jnp.int32),
        pltpu.VMEM((128, H), jnp.float32),
        pltpu.VMEM((ZR, H), jnp.float32),
        pltpu.SemaphoreType.DMA,
        pltpu.SemaphoreType.DMA,
    ],
    compiler_params=pltpu.CompilerParams(use_tc_tiling_on_sc=False),
)(_cnt_body)


def _dense_body(agg_ref, cnt_ref, xd_ref, wl_ref, bl_ref, wr_ref, out_ref,
                *, col):
    cnt = jnp.maximum(cnt_ref[:, col:col + 1], 1.0)
    agg = agg_ref[...] / cnt
    o = (jnp.dot(agg, wl_ref[...], preferred_element_type=jnp.float32)
         + bl_ref[...]
         + jnp.dot(xd_ref[...], wr_ref[...], preferred_element_type=jnp.float32))
    nrm = jnp.sqrt(jnp.sum(o * o, axis=-1, keepdims=True))
    h = o / jnp.maximum(nrm, 1e-12)
    out_ref[...] = jnp.maximum(h, 0.0)


def _dense_final_body(agg_ref, cnt_ref, xd_ref, wl_ref, bl_ref, wr_ref,
                      wo_ref, bo_ref, out_ref, *, col):
    cnt = jnp.maximum(cnt_ref[:, col:col + 1], 1.0)
    agg = agg_ref[...] / cnt
    o = (jnp.dot(agg, wl_ref[...], preferred_element_type=jnp.float32)
         + bl_ref[...]
         + jnp.dot(xd_ref[...], wr_ref[...], preferred_element_type=jnp.float32))
    nrm = jnp.sqrt(jnp.sum(o * o, axis=-1, keepdims=True))
    h = jnp.maximum(o / jnp.maximum(nrm, 1e-12), 0.0)
    e = jnp.dot(h, wo_ref[...], preferred_element_type=jnp.float32) + bo_ref[...]
    nrm2 = jnp.sqrt(jnp.sum(e * e, axis=-1, keepdims=True))
    out_ref[...] = e / jnp.maximum(nrm2, 1e-12)


_ROW_SPEC = pl.BlockSpec((BR, D), lambda i: (i, 0))
_W_SPEC = pl.BlockSpec((D, D), lambda i: (0, 0))
_B_SPEC = pl.BlockSpec((1, D), lambda i: (0, 0))


def _dense(agg, cnt, xd, wl, bl, wr, col):
    return pl.pallas_call(
        functools.partial(_dense_body, col=col),
        grid=(NP // BR,),
        in_specs=[_ROW_SPEC, _ROW_SPEC, _ROW_SPEC, _W_SPEC, _B_SPEC, _W_SPEC],
        out_specs=_ROW_SPEC,
        out_shape=jax.ShapeDtypeStruct((NP, D), jnp.float32),
    )(agg, cnt, xd, wl, bl.reshape(1, D), wr)


def _dense_final(agg, cnt, xd, wl, bl, wr, wo, bo, col):
    return pl.pallas_call(
        functools.partial(_dense_final_body, col=col),
        grid=(NP // BR,),
        in_specs=[_ROW_SPEC, _ROW_SPEC, _ROW_SPEC, _W_SPEC, _B_SPEC, _W_SPEC,
                  _W_SPEC, _B_SPEC],
        out_specs=_ROW_SPEC,
        out_shape=jax.ShapeDtypeStruct((N, D), jnp.float32),
    )(agg, cnt, xd, wl, bl.reshape(1, D), wr, wo, bo.reshape(1, D))


def kernel(x_user, x_item, edge_index,
           Wl_ui0, bl_ui0, Wr_ui0, Wl_iu0, bl_iu0, Wr_iu0,
           Wl_ui1, bl_ui1, Wr_ui1, Wl_iu1, bl_iu1, Wr_iu1,
           Wu, bu, Wi, bi):
    src = edge_index[0].astype(jnp.int32)
    dst = edge_index[1].astype(jnp.int32)
    pad = jnp.full((EP - E,), N, jnp.int32)
    src_p = jnp.concatenate([src, pad]).reshape(G, 128)
    dst_p = jnp.concatenate([dst, pad]).reshape(G, 128)
    xu = jnp.pad(x_user, ((0, NP - N), (0, 0)))
    xi = jnp.pad(x_item, ((0, NP - N), (0, 0)))

    cnt = _cnt_call(src_p, dst_p)

    agg_i0 = _agg_call(src_p, dst_p, xu.reshape(2 * NP, H))
    agg_u0 = _agg_call(dst_p, src_p, xi.reshape(2 * NP, H))
    item_h = _dense(agg_i0, cnt, xi, Wl_ui0, bl_ui0, Wr_ui0, 16)
    user_h = _dense(agg_u0, cnt, xu, Wl_iu0, bl_iu0, Wr_iu0, 0)

    agg_i1 = _agg_call(src_p, dst_p, user_h.reshape(2 * NP, H))
    agg_u1 = _agg_call(dst_p, src_p, item_h.reshape(2 * NP, H))
    item_emb = _dense_final(agg_i1, cnt, item_h, Wl_ui1, bl_ui1, Wr_ui1,
                            Wi, bi, 16)
    user_emb = _dense_final(agg_u1, cnt, user_h, Wl_iu1, bl_iu1, Wr_iu1,
                            Wu, bu, 0)
    return (user_emb, item_emb)


# speedup vs baseline: 1.5057x; 1.0274x over previous
import functools

import jax
import jax.numpy as jnp
from jax import lax
from jax.experimental import pallas as pl
from jax.experimental.pallas import tpu as pltpu
from jax.experimental.pallas import tpu_sc as plsc

N = 100000
E = 1600000
D = 32
H = 16

NSUB = 16
KB = 4
CHUNK = KB * 128

NP = 100352
TR = NP // NSUB
ZR = 128
EP = 1605632
G = EP // 128
GT = G // NSUB
NCHUNK = GT // KB

BR = 2048
_MESH = plsc.VectorSubcoreMesh(
    core_axis_name="c", subcore_axis_name="s", num_cores=2, num_subcores=NSUB
)


def _agg_body(gidx_hbm, sidx_hbm, table_hbm, out_hbm,
              acc_sh, gi_v, si_v, rows_v, zb_v, sem0, sem1,
              semi0, semi1, semi2, semi3):
    c = lax.axis_index("c")
    s = lax.axis_index("s")
    sems = (sem0, sem1)
    semi = (semi0, semi1, semi2, semi3)

    def _zero(j, carry):
        zb_v[j, :] = jnp.zeros((16,), jnp.float32)
        return carry

    lax.fori_loop(0, ZR, _zero, 0)
    r0 = s * TR
    for t in range(TR // ZR):
        pltpu.sync_copy(zb_v, acc_sh.at[pl.ds(r0 + t * ZR, ZR)])
    plsc.subcore_barrier()

    cvec = jnp.full((16,), c, jnp.int32)

    def _load_idx(sl, k):
        g0 = s * GT + k * KB
        pltpu.async_copy(gidx_hbm.at[pl.ds(g0, KB)], gi_v.at[sl], semi[sl])
        pltpu.async_copy(sidx_hbm.at[pl.ds(g0, KB)], si_v.at[sl], semi[sl])

    def _drain_idx(sl):
        pltpu.make_async_copy(gidx_hbm.at[pl.ds(0, KB)], gi_v.at[sl],
                              semi[sl]).wait()
        pltpu.make_async_copy(sidx_hbm.at[pl.ds(0, KB)], si_v.at[sl],
                              semi[sl]).wait()

    def _transform_fire(sl, b):
        for j in range(KB):
            for q in range(8):
                v = gi_v[sl, j, pl.ds(q * 16, 16)]
                gi_v[sl, j, pl.ds(q * 16, 16)] = v + v + cvec
        for j in range(KB):
            pltpu.async_copy(table_hbm.at[gi_v.at[sl, j]],
                             rows_v.at[b, pl.ds(j * 128, 128)], sems[b])

    def _drain(b):
        for j in range(KB):
            pltpu.make_async_copy(table_hbm.at[gi_v.at[0, j]],
                                  rows_v.at[b, pl.ds(j * 128, 128)],
                                  sems[b]).wait()

    def _scatter(sl, b):
        for j in range(KB):
            pltpu.sync_copy(rows_v.at[b, pl.ds(j * 128, 128)],
                            acc_sh.at[si_v.at[sl, j]], add=True)

    for sl in range(4):
        _load_idx(sl, sl)
    _drain_idx(0)
    _transform_fire(0, 0)
    _drain_idx(1)
    _transform_fire(1, 1)

    def _quad(it, carry):
        k = it * 4
        for m in range(4):
            b = m % 2
            sl_fire = (m + 2) % 4
            _drain(b)
            _scatter(m, b)

            @pl.when(k + 4 + m < NCHUNK)
            def _():
                _load_idx(m, k + 4 + m)

            @pl.when(k + 2 + m < NCHUNK)
            def _():
                _drain_idx(sl_fire)
                _transform_fire(sl_fire, b)

        return carry

    lax.fori_loop(0, NCHUNK // 4, _quad, 0)
    plsc.subcore_barrier()

    @pl.when(c == 0)
    def _():
        pltpu.sync_copy(acc_sh.at[pl.ds(r0, TR)],
                        out_hbm.at[pl.ds(r0, TR), pl.ds(0, 16)])

    @pl.when(c == 1)
    def _():
        pltpu.sync_copy(acc_sh.at[pl.ds(r0, TR)],
                        out_hbm.at[pl.ds(r0, TR), pl.ds(16, 16)])


_agg_call = functools.partial(
    pl.kernel,
    out_type=jax.ShapeDtypeStruct((NP, D), jnp.float32),
    mesh=_MESH,
    scratch_types=[
        pltpu.VMEM_SHARED((NP, H), jnp.float32),
        pltpu.VMEM((4, KB, 128), jnp.int32),
        pltpu.VMEM((4, KB, 128), jnp.int32),
        pltpu.VMEM((2, CHUNK, H), jnp.float32),
        pltpu.VMEM((ZR, H), jnp.float32),
        pltpu.SemaphoreType.DMA,
        pltpu.SemaphoreType.DMA,
        pltpu.SemaphoreType.DMA,
        pltpu.SemaphoreType.DMA,
        pltpu.SemaphoreType.DMA,
        pltpu.SemaphoreType.DMA,
    ],
    compiler_params=pltpu.CompilerParams(use_tc_tiling_on_sc=False),
)(_agg_body)


def _cnt_body(su_hbm, sd_hbm, out_hbm, acc_sh, si_v, ones_v, zb_v,
              sem0, sem1):
    c = lax.axis_index("c")
    s = lax.axis_index("s")

    def _zero(j, carry):
        zb_v[j, :] = jnp.zeros((16,), jnp.float32)
        return carry

    lax.fori_loop(0, ZR, _zero, 0)

    def _one(j, carry):
        ones_v[j, :] = jnp.ones((16,), jnp.float32)
        return carry

    lax.fori_loop(0, 128, _one, 0)
    r0 = s * TR
    for t in range(TR // ZR):
        pltpu.sync_copy(zb_v, acc_sh.at[pl.ds(r0 + t * ZR, ZR)])
    plsc.subcore_barrier()

    sems = (sem0, sem1)

    def _fire(b, k):
        g0 = s * GT + k * KB

        @pl.when(c == 0)
        def _():
            pltpu.sync_copy(su_hbm.at[pl.ds(g0, KB)], si_v.at[b])

        @pl.when(c == 1)
        def _():
            pltpu.sync_copy(sd_hbm.at[pl.ds(g0, KB)], si_v.at[b])

        for j in range(KB):
            pltpu.async_copy(ones_v, acc_sh.at[si_v.at[b, j]], sems[b],
                             add=True)

    def _drain(b):
        for j in range(KB):
            pltpu.make_async_copy(ones_v, acc_sh.at[si_v.at[b, j]],
                                  sems[b]).wait()

    _fire(0, 0)

    def _pair(k2, carry):
        k = k2 * 2
        _fire(1, k + 1)
        _drain(0)

        @pl.when(k + 2 < NCHUNK)
        def _():
            _fire(0, k + 2)

        _drain(1)
        return carry

    lax.fori_loop(0, NCHUNK // 2, _pair, 0)
    plsc.subcore_barrier()

    @pl.when(c == 0)
    def _():
        pltpu.sync_copy(acc_sh.at[pl.ds(r0, TR)],
                        out_hbm.at[pl.ds(r0, TR), pl.ds(0, 16)])

    @pl.when(c == 1)
    def _():
        pltpu.sync_copy(acc_sh.at[pl.ds(r0, TR)],
                        out_hbm.at[pl.ds(r0, TR), pl.ds(16, 16)])


_cnt_call = functools.partial(
    pl.kernel,
    out_type=jax.ShapeDtypeStruct((NP, D), jnp.float32),
    mesh=_MESH,
    scratch_types=[
        pltpu.VMEM_SHARED((NP, H), jnp.float32),
        pltpu.VMEM((2, KB, 128), jnp.int32),
        pltpu.VMEM((128, H), jnp.float32),
        pltpu.VMEM((ZR, H), jnp.float32),
        pltpu.SemaphoreType.DMA,
        pltpu.SemaphoreType.DMA,
    ],
    compiler_params=pltpu.CompilerParams(use_tc_tiling_on_sc=False),
)(_cnt_body)


def _dense_body(agg_ref, cnt_ref, xd_ref, wl_ref, bl_ref, wr_ref, out_ref,
                *, col):
    cnt = jnp.maximum(cnt_ref[:, col:col + 1], 1.0)
    agg = agg_ref[...] / cnt
    o = (jnp.dot(agg, wl_ref[...], preferred_element_type=jnp.float32)
         + bl_ref[...]
         + jnp.dot(xd_ref[...], wr_ref[...], preferred_element_type=jnp.float32))
    nrm = jnp.sqrt(jnp.sum(o * o, axis=-1, keepdims=True))
    h = o / jnp.maximum(nrm, 1e-12)
    out_ref[...] = jnp.maximum(h, 0.0)


def _dense_final_body(agg_ref, cnt_ref, xd_ref, wl_ref, bl_ref, wr_ref,
                      wo_ref, bo_ref, out_ref, *, col):
    cnt = jnp.maximum(cnt_ref[:, col:col + 1], 1.0)
    agg = agg_ref[...] / cnt
    o = (jnp.dot(agg, wl_ref[...], preferred_element_type=jnp.float32)
         + bl_ref[...]
         + jnp.dot(xd_ref[...], wr_ref[...], preferred_element_type=jnp.float32))
    nrm = jnp.sqrt(jnp.sum(o * o, axis=-1, keepdims=True))
    h = jnp.maximum(o / jnp.maximum(nrm, 1e-12), 0.0)
    e = jnp.dot(h, wo_ref[...], preferred_element_type=jnp.float32) + bo_ref[...]
    nrm2 = jnp.sqrt(jnp.sum(e * e, axis=-1, keepdims=True))
    out_ref[...] = e / jnp.maximum(nrm2, 1e-12)


_ROW_SPEC = pl.BlockSpec((BR, D), lambda i: (i, 0))
_W_SPEC = pl.BlockSpec((D, D), lambda i: (0, 0))
_B_SPEC = pl.BlockSpec((1, D), lambda i: (0, 0))


def _dense(agg, cnt, xd, wl, bl, wr, col):
    return pl.pallas_call(
        functools.partial(_dense_body, col=col),
        grid=(NP // BR,),
        in_specs=[_ROW_SPEC, _ROW_SPEC, _ROW_SPEC, _W_SPEC, _B_SPEC, _W_SPEC],
        out_specs=_ROW_SPEC,
        out_shape=jax.ShapeDtypeStruct((NP, D), jnp.float32),
    )(agg, cnt, xd, wl, bl.reshape(1, D), wr)


def _dense_final(agg, cnt, xd, wl, bl, wr, wo, bo, col):
    return pl.pallas_call(
        functools.partial(_dense_final_body, col=col),
        grid=(NP // BR,),
        in_specs=[_ROW_SPEC, _ROW_SPEC, _ROW_SPEC, _W_SPEC, _B_SPEC, _W_SPEC,
                  _W_SPEC, _B_SPEC],
        out_specs=_ROW_SPEC,
        out_shape=jax.ShapeDtypeStruct((N, D), jnp.float32),
    )(agg, cnt, xd, wl, bl.reshape(1, D), wr, wo, bo.reshape(1, D))


def kernel(x_user, x_item, edge_index,
           Wl_ui0, bl_ui0, Wr_ui0, Wl_iu0, bl_iu0, Wr_iu0,
           Wl_ui1, bl_ui1, Wr_ui1, Wl_iu1, bl_iu1, Wr_iu1,
           Wu, bu, Wi, bi):
    src = edge_index[0].astype(jnp.int32)
    dst = edge_index[1].astype(jnp.int32)
    pad = jnp.full((EP - E,), N, jnp.int32)
    src_p = jnp.concatenate([src, pad]).reshape(G, 128)
    dst_p = jnp.concatenate([dst, pad]).reshape(G, 128)
    xu = jnp.pad(x_user, ((0, NP - N), (0, 0)))
    xi = jnp.pad(x_item, ((0, NP - N), (0, 0)))

    cnt = _cnt_call(src_p, dst_p)

    agg_i0 = _agg_call(src_p, dst_p, xu.reshape(2 * NP, H))
    agg_u0 = _agg_call(dst_p, src_p, xi.reshape(2 * NP, H))
    item_h = _dense(agg_i0, cnt, xi, Wl_ui0, bl_ui0, Wr_ui0, 16)
    user_h = _dense(agg_u0, cnt, xu, Wl_iu0, bl_iu0, Wr_iu0, 0)

    agg_i1 = _agg_call(src_p, dst_p, user_h.reshape(2 * NP, H))
    agg_u1 = _agg_call(dst_p, src_p, item_h.reshape(2 * NP, H))
    item_emb = _dense_final(agg_i1, cnt, item_h, Wl_ui1, bl_ui1, Wr_ui1,
                            Wi, bi, 16)
    user_emb = _dense_final(agg_u1, cnt, user_h, Wl_iu1, bl_iu1, Wr_iu1,
                            Wu, bu, 0)
    return (user_emb, item_emb)
